# baseline (speedup 1.0000x reference)
import functools

import jax
import jax.numpy as jnp
from jax import lax
from jax.experimental import pallas as pl
from jax.experimental.pallas import tpu as pltpu
from jax.experimental.pallas import tpu_sc as plsc

E = 8
D_IN = 1024
D_OUT = 1024
S = 2048

BLK = 128
NBLK = (2 * S) // BLK + E
CAP = NBLK * BLK

NC = 2
NS = 16
NW = NC * NS
TPW = S // NW
CHUNK = 32

_NEG_INF = float("-inf")


def _router_body(x_ref, wg_ref, pos1_ref, pos2_ref, g1_ref, g2_ref,
                 be_ref, ba_ref, xi_ref, xpk_ref):
    x = x_ref[...]
    xu = pltpu.bitcast(x, jnp.uint32)
    rnd = ((xu >> 16) & 1) + 0x7FFF
    xb = (xu + rnd) >> 16
    lo = xb[:, :D_IN // 2]
    hi = xb[:, D_IN // 2:]
    xpk_ref[...] = pltpu.bitcast(lo | (hi << 16), jnp.int32)
    logits = jnp.dot(x, wg_ref[...], preferred_element_type=jnp.float32)
    e_iota = lax.broadcasted_iota(jnp.int32, (S, E), 1)
    m1 = jnp.max(logits, axis=-1, keepdims=True)
    i1 = jnp.min(jnp.where(logits == m1, e_iota, E), axis=-1, keepdims=True)
    masked = jnp.where(e_iota == i1, _NEG_INF, logits)
    m2 = jnp.max(masked, axis=-1, keepdims=True)
    i2 = jnp.min(jnp.where(masked == m2, e_iota, E), axis=-1, keepdims=True)
    t = jnp.exp(m2 - m1)
    g1_ref[...] = (1.0 / (1.0 + t)).reshape(NW, TPW)
    g2_ref[...] = (t / (1.0 + t)).reshape(NW, TPW)

    oh1 = (e_iota == i1).astype(jnp.float32)
    oh2 = (e_iota == i2).astype(jnp.float32)
    oh = jnp.concatenate([oh1, oh2], axis=1)
    TB = 256
    r_iota = lax.broadcasted_iota(jnp.int32, (TB, TB), 0)
    c_iota = lax.broadcasted_iota(jnp.int32, (TB, TB), 1)
    tril = (c_iota < r_iota).astype(jnp.float32)
    running = jnp.zeros((1, 2 * E), jnp.float32)
    ranks_parts = []
    for tb in range(S // TB):
        blk = oh[tb * TB:(tb + 1) * TB, :]
        ranks_parts.append(
            jnp.dot(tril, blk, preferred_element_type=jnp.float32) + running)
        running = running + jnp.sum(blk, axis=0, keepdims=True)
    ranks = jnp.concatenate(ranks_parts, axis=0)
    counts = running
    c1 = counts[:, :E]
    c = c1 + counts[:, E:]
    padded = jnp.ceil(c * (1.0 / BLK)) * BLK
    u8r = lax.broadcasted_iota(jnp.int32, (E, E), 0)
    u8c = lax.broadcasted_iota(jnp.int32, (E, E), 1)
    triu8 = (u8r < u8c).astype(jnp.float32)
    base = jnp.dot(padded, triu8, preferred_element_type=jnp.float32)

    rank1 = jnp.sum(oh1 * ranks[:, :E], axis=1, keepdims=True)
    rank2 = jnp.sum(oh2 * ranks[:, E:], axis=1, keepdims=True)
    base1 = jnp.sum(oh1 * base, axis=1, keepdims=True)
    base2 = jnp.sum(oh2 * base, axis=1, keepdims=True)
    c1sel = jnp.sum(oh2 * c1, axis=1, keepdims=True)
    pos1_ref[...] = (base1 + rank1).astype(jnp.int32).reshape(NW, TPW)
    pos2_ref[...] = (base2 + c1sel + rank2).astype(jnp.int32).reshape(NW, TPW)

    ends = base + padded
    total = jnp.max(ends, axis=-1, keepdims=True)
    b_iota = lax.broadcasted_iota(jnp.int32, (NBLK, 1), 0)
    starts = (b_iota * BLK).astype(jnp.float32)
    be = jnp.sum((starts >= ends).astype(jnp.int32), axis=1, keepdims=True)
    active = (starts < total).astype(jnp.int32)
    e_row = lax.broadcasted_iota(jnp.int32, (1, E), 1)
    lastexp = jnp.max(jnp.where(padded > 0, e_row, 0), axis=1, keepdims=True)
    be_ref[...] = jnp.where(active > 0, be, lastexp)
    ba_ref[...] = active
    lastblk = jnp.sum(active, axis=0, keepdims=True) - 1
    xi_ref[...] = jnp.where(active > 0, b_iota, lastblk)


def _router(x2, Wg):
    return pl.pallas_call(
        _router_body,
        out_shape=(
            jax.ShapeDtypeStruct((NW, TPW), jnp.int32),
            jax.ShapeDtypeStruct((NW, TPW), jnp.int32),
            jax.ShapeDtypeStruct((NW, TPW), jnp.float32),
            jax.ShapeDtypeStruct((NW, TPW), jnp.float32),
            jax.ShapeDtypeStruct((NBLK, 1), jnp.int32),
            jax.ShapeDtypeStruct((NBLK, 1), jnp.int32),
            jax.ShapeDtypeStruct((NBLK, 1), jnp.int32),
            jax.ShapeDtypeStruct((S, D_IN // 2), jnp.int32),
        ),
    )(x2, Wg)


@functools.cache
def _make_dispatch():
    mesh = plsc.VectorSubcoreMesh(core_axis_name="c", subcore_axis_name="s")

    @functools.partial(
        pl.kernel,
        out_type=jax.ShapeDtypeStruct((CAP, D_IN // 2), jnp.int32),
        mesh=mesh,
        scratch_types=[
            pltpu.VMEM((2, TPW), jnp.int32),
            pltpu.VMEM((TPW, D_IN // 2), jnp.int32),
            pltpu.SemaphoreType.DMA,
        ],
    )
    def _dispatch(pos1_hbm, pos2_hbm, xb_hbm, xs_hbm, idx_v, rows_v, sem):
        wid = lax.axis_index("s") * NC + lax.axis_index("c")
        pltpu.sync_copy(pos1_hbm.at[wid], idx_v.at[0])
        pltpu.sync_copy(pos2_hbm.at[wid], idx_v.at[1])
        pltpu.sync_copy(xb_hbm.at[pl.ds(wid * TPW, TPW)], rows_v)
        h0 = pltpu.async_copy(rows_v, xs_hbm.at[idx_v.at[0]], sem)
        h1 = pltpu.async_copy(rows_v, xs_hbm.at[idx_v.at[1]], sem)
        h0.wait()
        h1.wait()

    return _dispatch


def _gmm_body(be_ref, ba_ref, xi_ref, xs_ref, wg_ref, wu_ref, y_ref):
    b = pl.program_id(0)

    @pl.when(ba_ref[b, 0] > 0)
    def _():
        w32 = xs_ref[...]
        xlo = pltpu.bitcast((w32 & 0xFFFF).astype(jnp.uint16), jnp.bfloat16)
        xhi = pltpu.bitcast(
            ((w32 >> 16) & 0xFFFF).astype(jnp.uint16), jnp.bfloat16)
        H = D_IN // 2
        wg = wg_ref[0].astype(jnp.bfloat16)
        wu = wu_ref[0].astype(jnp.bfloat16)
        hg = (jnp.dot(xlo, wg[:H], preferred_element_type=jnp.float32)
              + jnp.dot(xhi, wg[H:], preferred_element_type=jnp.float32))
        hu = (jnp.dot(xlo, wu[:H], preferred_element_type=jnp.float32)
              + jnp.dot(xhi, wu[H:], preferred_element_type=jnp.float32))
        y_ref[...] = (hg * jax.nn.sigmoid(hg)) * hu


def _gmm(be, ba, xi, x_sorted, W_gate, W_up):
    grid_spec = pltpu.PrefetchScalarGridSpec(
        num_scalar_prefetch=3,
        grid=(NBLK,),
        in_specs=[
            pl.BlockSpec((BLK, D_IN // 2),
                         lambda b, be, ba, xi: (xi[b, 0], 0)),
            pl.BlockSpec((1, D_IN, D_OUT),
                         lambda b, be, ba, xi: (be[b, 0], 0, 0)),
            pl.BlockSpec((1, D_IN, D_OUT),
                         lambda b, be, ba, xi: (be[b, 0], 0, 0)),
        ],
        out_specs=pl.BlockSpec((BLK, D_OUT),
                               lambda b, be, ba, xi: (xi[b, 0], 0)),
    )
    return pl.pallas_call(
        _gmm_body,
        grid_spec=grid_spec,
        out_shape=jax.ShapeDtypeStruct((CAP, D_OUT), jnp.float32),
    )(be, ba, xi, x_sorted, W_gate, W_up)


@functools.cache
def _make_combine():
    mesh = plsc.VectorSubcoreMesh(core_axis_name="c", subcore_axis_name="s")

    CH = 16
    NCH = TPW // CH

    @functools.partial(
        pl.kernel,
        out_type=jax.ShapeDtypeStruct((S, D_OUT), jnp.float32),
        mesh=mesh,
        scratch_types=[
            pltpu.VMEM((TPW,), jnp.int32),
            pltpu.VMEM((TPW,), jnp.int32),
            pltpu.VMEM((TPW,), jnp.float32),
            pltpu.VMEM((TPW,), jnp.float32),
            pltpu.VMEM((CH, D_OUT), jnp.float32),
            pltpu.VMEM((CH, D_OUT), jnp.float32),
            pltpu.VMEM((CH, D_OUT), jnp.float32),
            pltpu.VMEM((CH, D_OUT), jnp.float32),
            pltpu.VMEM((CH, D_OUT), jnp.float32),
            pltpu.VMEM((CH, D_OUT), jnp.float32),
            pltpu.SemaphoreType.DMA,
            pltpu.SemaphoreType.DMA,
            pltpu.SemaphoreType.DMA,
            pltpu.SemaphoreType.DMA,
        ],
    )
    def _combine(pos1_hbm, pos2_hbm, g1_hbm, g2_hbm, y_hbm, out_hbm,
                 i1v, i2v, g1v, g2v, r1a, r2a, r1b, r2b, ova, ovb,
                 sem0, sem1, osem, osem2):
        wid = lax.axis_index("s") * NC + lax.axis_index("c")
        pltpu.sync_copy(pos1_hbm.at[wid], i1v)
        pltpu.sync_copy(pos2_hbm.at[wid], i2v)
        pltpu.sync_copy(g1_hbm.at[wid], g1v)
        pltpu.sync_copy(g2_hbm.at[wid], g2v)
        bufs = ((r1a, r2a, sem0), (r1b, r2b, sem1))

        def issue(ch):
            b1, b2, sem = bufs[ch % 2]
            ha = pltpu.async_copy(y_hbm.at[i1v.at[pl.ds(ch * CH, CH)]], b1,
                                  sem)
            hb = pltpu.async_copy(y_hbm.at[i2v.at[pl.ds(ch * CH, CH)]], b2,
                                  sem)
            return ha, hb

        hs = issue(0)
        ovs = (ova, ovb)
        osems = (osem, osem2)
        ohs = [None, None]
        for ch in range(NCH):
            b1, b2, _ = bufs[ch % 2]
            hsn = issue(ch + 1) if ch + 1 < NCH else None
            hs[0].wait()
            hs[1].wait()
            hs = hsn
            ov = ovs[ch % 2]
            if ohs[ch % 2] is not None:
                ohs[ch % 2].wait()
            gveca = g1v[pl.ds(ch * CH, 16)]
            gvecb = g2v[pl.ds(ch * CH, 16)]

            def col_body(j, carry):
                sl = pl.ds(16 * j, 16)
                for r in range(CH):
                    ov[r, sl] = (gveca[r] * b1[r, sl]
                                 + gvecb[r] * b2[r, sl])
                return carry

            lax.fori_loop(0, D_OUT // 16, col_body, 0)
            ohs[ch % 2] = pltpu.async_copy(
                ov, out_hbm.at[pl.ds(wid * TPW + ch * CH, CH)],
                osems[ch % 2])
        ohs[0].wait()
        ohs[1].wait()

    return _combine


def kernel(x, Wg, W_gate, W_up):
    B = x.shape[0]
    x2 = x.reshape(B * S, D_IN)
    pos1, pos2, g1, g2, be, ba, xi, xpk = _router(x2, Wg)
    x_sorted = _make_dispatch()(pos1, pos2, xpk)
    y_sorted = _gmm(be, ba, xi, x_sorted, W_gate, W_up)
    out = _make_combine()(pos1, pos2, g1, g2, y_sorted)
    return out.reshape(B, S, D_OUT)


# speedup vs baseline: 1.0661x; 1.0661x over previous
import functools

import jax
import jax.numpy as jnp
from jax import lax
from jax.experimental import pallas as pl
from jax.experimental.pallas import tpu as pltpu
from jax.experimental.pallas import tpu_sc as plsc

E = 8
D_IN = 1024
D_OUT = 1024
S = 2048

BLK = 256
NBLK = (2 * S) // BLK + E
CAP = NBLK * BLK

NC = 2
NS = 16
NW = NC * NS
TPW = S // NW
CHUNK = 32

_NEG_INF = float("-inf")


def _router_body(x_ref, wg_ref, pos1_ref, pos2_ref, g1_ref, g2_ref,
                 be_ref, ba_ref, xi_ref, xpk_ref):
    x = x_ref[...]
    xu = pltpu.bitcast(x, jnp.uint32)
    rnd = ((xu >> 16) & 1) + 0x7FFF
    xb = (xu + rnd) >> 16
    lo = xb[:, :D_IN // 2]
    hi = xb[:, D_IN // 2:]
    xpk_ref[...] = pltpu.bitcast(lo | (hi << 16), jnp.int32)
    logits = jnp.dot(x, wg_ref[...], preferred_element_type=jnp.float32)
    e_iota = lax.broadcasted_iota(jnp.int32, (S, E), 1)
    m1 = jnp.max(logits, axis=-1, keepdims=True)
    i1 = jnp.min(jnp.where(logits == m1, e_iota, E), axis=-1, keepdims=True)
    masked = jnp.where(e_iota == i1, _NEG_INF, logits)
    m2 = jnp.max(masked, axis=-1, keepdims=True)
    i2 = jnp.min(jnp.where(masked == m2, e_iota, E), axis=-1, keepdims=True)
    t = jnp.exp(m2 - m1)
    g1_ref[...] = (1.0 / (1.0 + t)).reshape(NW, TPW)
    g2_ref[...] = (t / (1.0 + t)).reshape(NW, TPW)

    oh1 = (e_iota == i1).astype(jnp.float32)
    oh2 = (e_iota == i2).astype(jnp.float32)
    oh = jnp.concatenate([oh1, oh2], axis=1)
    TB = 256
    r_iota = lax.broadcasted_iota(jnp.int32, (TB, TB), 0)
    c_iota = lax.broadcasted_iota(jnp.int32, (TB, TB), 1)
    tril = (c_iota < r_iota).astype(jnp.float32)
    running = jnp.zeros((1, 2 * E), jnp.float32)
    ranks_parts = []
    for tb in range(S // TB):
        blk = oh[tb * TB:(tb + 1) * TB, :]
        ranks_parts.append(
            jnp.dot(tril, blk, preferred_element_type=jnp.float32) + running)
        running = running + jnp.sum(blk, axis=0, keepdims=True)
    ranks = jnp.concatenate(ranks_parts, axis=0)
    counts = running
    c1 = counts[:, :E]
    c = c1 + counts[:, E:]
    padded = jnp.ceil(c * (1.0 / BLK)) * BLK
    u8r = lax.broadcasted_iota(jnp.int32, (E, E), 0)
    u8c = lax.broadcasted_iota(jnp.int32, (E, E), 1)
    triu8 = (u8r < u8c).astype(jnp.float32)
    base = jnp.dot(padded, triu8, preferred_element_type=jnp.float32)

    rank1 = jnp.sum(oh1 * ranks[:, :E], axis=1, keepdims=True)
    rank2 = jnp.sum(oh2 * ranks[:, E:], axis=1, keepdims=True)
    base1 = jnp.sum(oh1 * base, axis=1, keepdims=True)
    base2 = jnp.sum(oh2 * base, axis=1, keepdims=True)
    c1sel = jnp.sum(oh2 * c1, axis=1, keepdims=True)
    pos1_ref[...] = (base1 + rank1).astype(jnp.int32).reshape(NW, TPW)
    pos2_ref[...] = (base2 + c1sel + rank2).astype(jnp.int32).reshape(NW, TPW)

    ends = base + padded
    total = jnp.max(ends, axis=-1, keepdims=True)
    b_iota = lax.broadcasted_iota(jnp.int32, (NBLK, 1), 0)
    starts = (b_iota * BLK).astype(jnp.float32)
    be = jnp.sum((starts >= ends).astype(jnp.int32), axis=1, keepdims=True)
    active = (starts < total).astype(jnp.int32)
    e_row = lax.broadcasted_iota(jnp.int32, (1, E), 1)
    lastexp = jnp.max(jnp.where(padded > 0, e_row, 0), axis=1, keepdims=True)
    be_ref[...] = jnp.where(active > 0, be, lastexp)
    ba_ref[...] = active
    lastblk = jnp.sum(active, axis=0, keepdims=True) - 1
    xi_ref[...] = jnp.where(active > 0, b_iota, lastblk)


def _router(x2, Wg):
    return pl.pallas_call(
        _router_body,
        out_shape=(
            jax.ShapeDtypeStruct((NW, TPW), jnp.int32),
            jax.ShapeDtypeStruct((NW, TPW), jnp.int32),
            jax.ShapeDtypeStruct((NW, TPW), jnp.float32),
            jax.ShapeDtypeStruct((NW, TPW), jnp.float32),
            jax.ShapeDtypeStruct((NBLK, 1), jnp.int32),
            jax.ShapeDtypeStruct((NBLK, 1), jnp.int32),
            jax.ShapeDtypeStruct((NBLK, 1), jnp.int32),
            jax.ShapeDtypeStruct((S, D_IN // 2), jnp.int32),
        ),
    )(x2, Wg)


@functools.cache
def _make_dispatch():
    mesh = plsc.VectorSubcoreMesh(core_axis_name="c", subcore_axis_name="s")

    @functools.partial(
        pl.kernel,
        out_type=jax.ShapeDtypeStruct((CAP, D_IN // 2), jnp.int32),
        mesh=mesh,
        scratch_types=[
            pltpu.VMEM((2, TPW), jnp.int32),
            pltpu.VMEM((TPW, D_IN // 2), jnp.int32),
            pltpu.SemaphoreType.DMA,
        ],
    )
    def _dispatch(pos1_hbm, pos2_hbm, xb_hbm, xs_hbm, idx_v, rows_v, sem):
        wid = lax.axis_index("s") * NC + lax.axis_index("c")
        pltpu.sync_copy(pos1_hbm.at[wid], idx_v.at[0])
        pltpu.sync_copy(pos2_hbm.at[wid], idx_v.at[1])
        pltpu.sync_copy(xb_hbm.at[pl.ds(wid * TPW, TPW)], rows_v)
        h0 = pltpu.async_copy(rows_v, xs_hbm.at[idx_v.at[0]], sem)
        h1 = pltpu.async_copy(rows_v, xs_hbm.at[idx_v.at[1]], sem)
        h0.wait()
        h1.wait()

    return _dispatch


def _gmm_body(be_ref, ba_ref, xi_ref, xs_ref, wg_ref, wu_ref, y_ref):
    b = pl.program_id(0)

    @pl.when(ba_ref[b, 0] > 0)
    def _():
        w32 = xs_ref[...]
        xlo = pltpu.bitcast((w32 & 0xFFFF).astype(jnp.uint16), jnp.bfloat16)
        xhi = pltpu.bitcast(
            ((w32 >> 16) & 0xFFFF).astype(jnp.uint16), jnp.bfloat16)
        H = D_IN // 2
        wg = wg_ref[0].astype(jnp.bfloat16)
        wu = wu_ref[0].astype(jnp.bfloat16)
        hg = (jnp.dot(xlo, wg[:H], preferred_element_type=jnp.float32)
              + jnp.dot(xhi, wg[H:], preferred_element_type=jnp.float32))
        hu = (jnp.dot(xlo, wu[:H], preferred_element_type=jnp.float32)
              + jnp.dot(xhi, wu[H:], preferred_element_type=jnp.float32))
        y_ref[...] = (hg * jax.nn.sigmoid(hg)) * hu


def _gmm(be, ba, xi, x_sorted, W_gate, W_up):
    grid_spec = pltpu.PrefetchScalarGridSpec(
        num_scalar_prefetch=3,
        grid=(NBLK,),
        in_specs=[
            pl.BlockSpec((BLK, D_IN // 2),
                         lambda b, be, ba, xi: (xi[b, 0], 0)),
            pl.BlockSpec((1, D_IN, D_OUT),
                         lambda b, be, ba, xi: (be[b, 0], 0, 0)),
            pl.BlockSpec((1, D_IN, D_OUT),
                         lambda b, be, ba, xi: (be[b, 0], 0, 0)),
        ],
        out_specs=pl.BlockSpec((BLK, D_OUT),
                               lambda b, be, ba, xi: (xi[b, 0], 0)),
    )
    return pl.pallas_call(
        _gmm_body,
        grid_spec=grid_spec,
        out_shape=jax.ShapeDtypeStruct((CAP, D_OUT), jnp.float32),
    )(be, ba, xi, x_sorted, W_gate, W_up)


@functools.cache
def _make_combine():
    mesh = plsc.VectorSubcoreMesh(core_axis_name="c", subcore_axis_name="s")

    CH = 16
    NCH = TPW // CH

    @functools.partial(
        pl.kernel,
        out_type=jax.ShapeDtypeStruct((S, D_OUT), jnp.float32),
        mesh=mesh,
        scratch_types=[
            pltpu.VMEM((TPW,), jnp.int32),
            pltpu.VMEM((TPW,), jnp.int32),
            pltpu.VMEM((TPW,), jnp.float32),
            pltpu.VMEM((TPW,), jnp.float32),
            pltpu.VMEM((CH, D_OUT), jnp.float32),
            pltpu.VMEM((CH, D_OUT), jnp.float32),
            pltpu.VMEM((CH, D_OUT), jnp.float32),
            pltpu.VMEM((CH, D_OUT), jnp.float32),
            pltpu.VMEM((CH, D_OUT), jnp.float32),
            pltpu.VMEM((CH, D_OUT), jnp.float32),
            pltpu.SemaphoreType.DMA,
            pltpu.SemaphoreType.DMA,
            pltpu.SemaphoreType.DMA,
            pltpu.SemaphoreType.DMA,
        ],
    )
    def _combine(pos1_hbm, pos2_hbm, g1_hbm, g2_hbm, y_hbm, out_hbm,
                 i1v, i2v, g1v, g2v, r1a, r2a, r1b, r2b, ova, ovb,
                 sem0, sem1, osem, osem2):
        wid = lax.axis_index("s") * NC + lax.axis_index("c")
        pltpu.sync_copy(pos1_hbm.at[wid], i1v)
        pltpu.sync_copy(pos2_hbm.at[wid], i2v)
        pltpu.sync_copy(g1_hbm.at[wid], g1v)
        pltpu.sync_copy(g2_hbm.at[wid], g2v)
        bufs = ((r1a, r2a, sem0), (r1b, r2b, sem1))

        def issue(ch):
            b1, b2, sem = bufs[ch % 2]
            ha = pltpu.async_copy(y_hbm.at[i1v.at[pl.ds(ch * CH, CH)]], b1,
                                  sem)
            hb = pltpu.async_copy(y_hbm.at[i2v.at[pl.ds(ch * CH, CH)]], b2,
                                  sem)
            return ha, hb

        hs = issue(0)
        ovs = (ova, ovb)
        osems = (osem, osem2)
        ohs = [None, None]
        for ch in range(NCH):
            b1, b2, _ = bufs[ch % 2]
            hsn = issue(ch + 1) if ch + 1 < NCH else None
            hs[0].wait()
            hs[1].wait()
            hs = hsn
            ov = ovs[ch % 2]
            if ohs[ch % 2] is not None:
                ohs[ch % 2].wait()
            gveca = g1v[pl.ds(ch * CH, 16)]
            gvecb = g2v[pl.ds(ch * CH, 16)]

            def col_body(j, carry):
                sl = pl.ds(16 * j, 16)
                for r in range(CH):
                    ov[r, sl] = (gveca[r] * b1[r, sl]
                                 + gvecb[r] * b2[r, sl])
                return carry

            lax.fori_loop(0, D_OUT // 16, col_body, 0)
            ohs[ch % 2] = pltpu.async_copy(
                ov, out_hbm.at[pl.ds(wid * TPW + ch * CH, CH)],
                osems[ch % 2])
        ohs[0].wait()
        ohs[1].wait()

    return _combine


def kernel(x, Wg, W_gate, W_up):
    B = x.shape[0]
    x2 = x.reshape(B * S, D_IN)
    pos1, pos2, g1, g2, be, ba, xi, xpk = _router(x2, Wg)
    x_sorted = _make_dispatch()(pos1, pos2, xpk)
    y_sorted = _gmm(be, ba, xi, x_sorted, W_gate, W_up)
    out = _make_combine()(pos1, pos2, g1, g2, y_sorted)
    return out.reshape(B, S, D_OUT)


# speedup vs baseline: 1.1406x; 1.0698x over previous
import functools

import jax
import jax.numpy as jnp
from jax import lax
from jax.experimental import pallas as pl
from jax.experimental.pallas import tpu as pltpu
from jax.experimental.pallas import tpu_sc as plsc

E = 8
D_IN = 1024
D_OUT = 1024
S = 2048

BLK = 512
NBLK = (2 * S) // BLK + E
CAP = NBLK * BLK

NC = 2
NS = 16
NW = NC * NS
TPW = S // NW
CHUNK = 32

_NEG_INF = float("-inf")


def _router_body(x_ref, wg_ref, pos1_ref, pos2_ref, g1_ref, g2_ref,
                 be_ref, ba_ref, xi_ref, xpk_ref):
    x = x_ref[...]
    xu = pltpu.bitcast(x, jnp.uint32)
    rnd = ((xu >> 16) & 1) + 0x7FFF
    xb = (xu + rnd) >> 16
    lo = xb[:, :D_IN // 2]
    hi = xb[:, D_IN // 2:]
    xpk_ref[...] = pltpu.bitcast(lo | (hi << 16), jnp.int32)
    logits = jnp.dot(x, wg_ref[...], preferred_element_type=jnp.float32)
    e_iota = lax.broadcasted_iota(jnp.int32, (S, E), 1)
    m1 = jnp.max(logits, axis=-1, keepdims=True)
    i1 = jnp.min(jnp.where(logits == m1, e_iota, E), axis=-1, keepdims=True)
    masked = jnp.where(e_iota == i1, _NEG_INF, logits)
    m2 = jnp.max(masked, axis=-1, keepdims=True)
    i2 = jnp.min(jnp.where(masked == m2, e_iota, E), axis=-1, keepdims=True)
    t = jnp.exp(m2 - m1)
    g1_ref[...] = (1.0 / (1.0 + t)).reshape(NW, TPW)
    g2_ref[...] = (t / (1.0 + t)).reshape(NW, TPW)

    oh1 = (e_iota == i1).astype(jnp.float32)
    oh2 = (e_iota == i2).astype(jnp.float32)
    oh = jnp.concatenate([oh1, oh2], axis=1)
    TB = 256
    r_iota = lax.broadcasted_iota(jnp.int32, (TB, TB), 0)
    c_iota = lax.broadcasted_iota(jnp.int32, (TB, TB), 1)
    tril = (c_iota < r_iota).astype(jnp.float32)
    running = jnp.zeros((1, 2 * E), jnp.float32)
    ranks_parts = []
    for tb in range(S // TB):
        blk = oh[tb * TB:(tb + 1) * TB, :]
        ranks_parts.append(
            jnp.dot(tril, blk, preferred_element_type=jnp.float32) + running)
        running = running + jnp.sum(blk, axis=0, keepdims=True)
    ranks = jnp.concatenate(ranks_parts, axis=0)
    counts = running
    c1 = counts[:, :E]
    c = c1 + counts[:, E:]
    padded = jnp.ceil(c * (1.0 / BLK)) * BLK
    u8r = lax.broadcasted_iota(jnp.int32, (E, E), 0)
    u8c = lax.broadcasted_iota(jnp.int32, (E, E), 1)
    triu8 = (u8r < u8c).astype(jnp.float32)
    base = jnp.dot(padded, triu8, preferred_element_type=jnp.float32)

    rank1 = jnp.sum(oh1 * ranks[:, :E], axis=1, keepdims=True)
    rank2 = jnp.sum(oh2 * ranks[:, E:], axis=1, keepdims=True)
    base1 = jnp.sum(oh1 * base, axis=1, keepdims=True)
    base2 = jnp.sum(oh2 * base, axis=1, keepdims=True)
    c1sel = jnp.sum(oh2 * c1, axis=1, keepdims=True)
    pos1_ref[...] = (base1 + rank1).astype(jnp.int32).reshape(NW, TPW)
    pos2_ref[...] = (base2 + c1sel + rank2).astype(jnp.int32).reshape(NW, TPW)

    ends = base + padded
    total = jnp.max(ends, axis=-1, keepdims=True)
    b_iota = lax.broadcasted_iota(jnp.int32, (NBLK, 1), 0)
    starts = (b_iota * BLK).astype(jnp.float32)
    be = jnp.sum((starts >= ends).astype(jnp.int32), axis=1, keepdims=True)
    active = (starts < total).astype(jnp.int32)
    e_row = lax.broadcasted_iota(jnp.int32, (1, E), 1)
    lastexp = jnp.max(jnp.where(padded > 0, e_row, 0), axis=1, keepdims=True)
    be_ref[...] = jnp.where(active > 0, be, lastexp)
    ba_ref[...] = active
    lastblk = jnp.sum(active, axis=0, keepdims=True) - 1
    xi_ref[...] = jnp.where(active > 0, b_iota, lastblk)


def _router(x2, Wg):
    return pl.pallas_call(
        _router_body,
        out_shape=(
            jax.ShapeDtypeStruct((NW, TPW), jnp.int32),
            jax.ShapeDtypeStruct((NW, TPW), jnp.int32),
            jax.ShapeDtypeStruct((NW, TPW), jnp.float32),
            jax.ShapeDtypeStruct((NW, TPW), jnp.float32),
            jax.ShapeDtypeStruct((NBLK, 1), jnp.int32),
            jax.ShapeDtypeStruct((NBLK, 1), jnp.int32),
            jax.ShapeDtypeStruct((NBLK, 1), jnp.int32),
            jax.ShapeDtypeStruct((S, D_IN // 2), jnp.int32),
        ),
    )(x2, Wg)


@functools.cache
def _make_dispatch():
    mesh = plsc.VectorSubcoreMesh(core_axis_name="c", subcore_axis_name="s")

    @functools.partial(
        pl.kernel,
        out_type=jax.ShapeDtypeStruct((CAP, D_IN // 2), jnp.int32),
        mesh=mesh,
        scratch_types=[
            pltpu.VMEM((2, TPW), jnp.int32),
            pltpu.VMEM((TPW, D_IN // 2), jnp.int32),
            pltpu.SemaphoreType.DMA,
        ],
    )
    def _dispatch(pos1_hbm, pos2_hbm, xb_hbm, xs_hbm, idx_v, rows_v, sem):
        wid = lax.axis_index("s") * NC + lax.axis_index("c")
        pltpu.sync_copy(pos1_hbm.at[wid], idx_v.at[0])
        pltpu.sync_copy(pos2_hbm.at[wid], idx_v.at[1])
        pltpu.sync_copy(xb_hbm.at[pl.ds(wid * TPW, TPW)], rows_v)
        h0 = pltpu.async_copy(rows_v, xs_hbm.at[idx_v.at[0]], sem)
        h1 = pltpu.async_copy(rows_v, xs_hbm.at[idx_v.at[1]], sem)
        h0.wait()
        h1.wait()

    return _dispatch


def _gmm_body(be_ref, ba_ref, xi_ref, xs_ref, wg_ref, wu_ref, y_ref):
    b = pl.program_id(0)

    @pl.when(ba_ref[b, 0] > 0)
    def _():
        w32 = xs_ref[...]
        xlo = pltpu.bitcast((w32 & 0xFFFF).astype(jnp.uint16), jnp.bfloat16)
        xhi = pltpu.bitcast(
            ((w32 >> 16) & 0xFFFF).astype(jnp.uint16), jnp.bfloat16)
        H = D_IN // 2
        wg = wg_ref[0].astype(jnp.bfloat16)
        wu = wu_ref[0].astype(jnp.bfloat16)
        hg = (jnp.dot(xlo, wg[:H], preferred_element_type=jnp.float32)
              + jnp.dot(xhi, wg[H:], preferred_element_type=jnp.float32))
        hu = (jnp.dot(xlo, wu[:H], preferred_element_type=jnp.float32)
              + jnp.dot(xhi, wu[H:], preferred_element_type=jnp.float32))
        y_ref[...] = (hg * jax.nn.sigmoid(hg)) * hu


def _gmm(be, ba, xi, x_sorted, W_gate, W_up):
    grid_spec = pltpu.PrefetchScalarGridSpec(
        num_scalar_prefetch=3,
        grid=(NBLK,),
        in_specs=[
            pl.BlockSpec((BLK, D_IN // 2),
                         lambda b, be, ba, xi: (xi[b, 0], 0)),
            pl.BlockSpec((1, D_IN, D_OUT),
                         lambda b, be, ba, xi: (be[b, 0], 0, 0)),
            pl.BlockSpec((1, D_IN, D_OUT),
                         lambda b, be, ba, xi: (be[b, 0], 0, 0)),
        ],
        out_specs=pl.BlockSpec((BLK, D_OUT),
                               lambda b, be, ba, xi: (xi[b, 0], 0)),
    )
    return pl.pallas_call(
        _gmm_body,
        grid_spec=grid_spec,
        out_shape=jax.ShapeDtypeStruct((CAP, D_OUT), jnp.float32),
    )(be, ba, xi, x_sorted, W_gate, W_up)


@functools.cache
def _make_combine():
    mesh = plsc.VectorSubcoreMesh(core_axis_name="c", subcore_axis_name="s")

    CH = 16
    NCH = TPW // CH

    @functools.partial(
        pl.kernel,
        out_type=jax.ShapeDtypeStruct((S, D_OUT), jnp.float32),
        mesh=mesh,
        scratch_types=[
            pltpu.VMEM((TPW,), jnp.int32),
            pltpu.VMEM((TPW,), jnp.int32),
            pltpu.VMEM((TPW,), jnp.float32),
            pltpu.VMEM((TPW,), jnp.float32),
            pltpu.VMEM((CH, D_OUT), jnp.float32),
            pltpu.VMEM((CH, D_OUT), jnp.float32),
            pltpu.VMEM((CH, D_OUT), jnp.float32),
            pltpu.VMEM((CH, D_OUT), jnp.float32),
            pltpu.VMEM((CH, D_OUT), jnp.float32),
            pltpu.VMEM((CH, D_OUT), jnp.float32),
            pltpu.SemaphoreType.DMA,
            pltpu.SemaphoreType.DMA,
            pltpu.SemaphoreType.DMA,
            pltpu.SemaphoreType.DMA,
        ],
    )
    def _combine(pos1_hbm, pos2_hbm, g1_hbm, g2_hbm, y_hbm, out_hbm,
                 i1v, i2v, g1v, g2v, r1a, r2a, r1b, r2b, ova, ovb,
                 sem0, sem1, osem, osem2):
        wid = lax.axis_index("s") * NC + lax.axis_index("c")
        pltpu.sync_copy(pos1_hbm.at[wid], i1v)
        pltpu.sync_copy(pos2_hbm.at[wid], i2v)
        pltpu.sync_copy(g1_hbm.at[wid], g1v)
        pltpu.sync_copy(g2_hbm.at[wid], g2v)
        bufs = ((r1a, r2a, sem0), (r1b, r2b, sem1))

        def issue(ch):
            b1, b2, sem = bufs[ch % 2]
            ha = pltpu.async_copy(y_hbm.at[i1v.at[pl.ds(ch * CH, CH)]], b1,
                                  sem)
            hb = pltpu.async_copy(y_hbm.at[i2v.at[pl.ds(ch * CH, CH)]], b2,
                                  sem)
            return ha, hb

        hs = issue(0)
        ovs = (ova, ovb)
        osems = (osem, osem2)
        ohs = [None, None]
        for ch in range(NCH):
            b1, b2, _ = bufs[ch % 2]
            hsn = issue(ch + 1) if ch + 1 < NCH else None
            hs[0].wait()
            hs[1].wait()
            hs = hsn
            ov = ovs[ch % 2]
            if ohs[ch % 2] is not None:
                ohs[ch % 2].wait()
            gveca = g1v[pl.ds(ch * CH, 16)]
            gvecb = g2v[pl.ds(ch * CH, 16)]

            def col_body(j, carry):
                sl = pl.ds(16 * j, 16)
                for r in range(CH):
                    ov[r, sl] = (gveca[r] * b1[r, sl]
                                 + gvecb[r] * b2[r, sl])
                return carry

            lax.fori_loop(0, D_OUT // 16, col_body, 0)
            ohs[ch % 2] = pltpu.async_copy(
                ov, out_hbm.at[pl.ds(wid * TPW + ch * CH, CH)],
                osems[ch % 2])
        ohs[0].wait()
        ohs[1].wait()

    return _combine


def kernel(x, Wg, W_gate, W_up):
    B = x.shape[0]
    x2 = x.reshape(B * S, D_IN)
    pos1, pos2, g1, g2, be, ba, xi, xpk = _router(x2, Wg)
    x_sorted = _make_dispatch()(pos1, pos2, xpk)
    y_sorted = _gmm(be, ba, xi, x_sorted, W_gate, W_up)
    out = _make_combine()(pos1, pos2, g1, g2, y_sorted)
    return out.reshape(B, S, D_OUT)
